# baseline (speedup 1.0000x reference)
import functools
import jax
import jax.numpy as jnp
from jax import lax
from jax.experimental import pallas as pl
from jax.experimental.pallas import tpu as pltpu
from jax.experimental.pallas import tpu_sc as plsc

_R = 16
_NX = 4
_NP = 2


def kernel(x, pe):
    batch, seq_len, d_model = x.shape
    n_rows = batch * seq_len
    x2 = x.reshape(n_rows, d_model)
    nw = 32
    seq_per_w = seq_len // nw
    n_chunks = seq_per_w // _R
    n_steps = n_chunks * batch

    mesh = plsc.VectorSubcoreMesh(
        core_axis_name="c", subcore_axis_name="s", num_cores=2, num_subcores=16
    )

    @functools.partial(
        pl.kernel,
        mesh=mesh,
        out_type=jax.ShapeDtypeStruct((n_rows, d_model), jnp.float32),
        scratch_types=[
            [pltpu.VMEM((_R, d_model), jnp.float32) for _ in range(_NX)],
            [pltpu.VMEM((_R, d_model), jnp.float32) for _ in range(_NP)],
            [pltpu.SemaphoreType.DMA for _ in range(_NX)],
            [pltpu.SemaphoreType.DMA for _ in range(_NP)],
            [pltpu.SemaphoreType.DMA for _ in range(_NX)],
        ],
    )
    def sc_add(x_hbm, pe_hbm, out_hbm, xb, peb, x_sems, pe_sems, st_sems):
        wid = lax.axis_index("s") * 2 + lax.axis_index("c")
        seq0 = wid * seq_per_w

        def xrow0(t):
            return (t % batch) * seq_len + seq0 + (t // batch) * _R

        def issue_x(t, sx):
            pltpu.async_copy(x_hbm.at[pl.ds(xrow0(t), _R)], xb[sx], x_sems[sx])

        def issue_pe(c, sp):
            pltpu.async_copy(
                pe_hbm.at[pl.ds(seq0 + c * _R, _R)], peb[sp], pe_sems[sp]
            )

        def wait(sems, slot, dst, src=None):
            pltpu.make_async_copy(
                x_hbm.at[pl.ds(0, _R)] if src is None else src, dst, sems[slot]
            ).wait()

        def compute(sx, sp):
            @pl.loop(0, _R)
            def row(r):
                @pl.loop(0, d_model // 16, unroll=8)
                def col(g):
                    sl = pl.ds(g * 16, 16)
                    xb[sx][r, sl] = xb[sx][r, sl] + peb[sp][r, sl]

        issue_pe(0, 0)
        issue_x(0, 0)
        issue_x(1, 1)

        @pl.loop(0, n_steps // 8)
        def outer(i):
            for u in range(8):
                t = i * 8 + u
                sx = u % _NX
                sp = u // 4
                c = t // batch

                if u % 4 == 0:
                    wait(pe_sems, sp, peb[sp])

                    @pl.when(c + 1 < n_chunks)
                    def _():
                        issue_pe(c + 1, (sp + 1) % _NP)

                wait(x_sems, sx, xb[sx])
                compute(sx, sp)
                pltpu.async_copy(
                    xb[sx], out_hbm.at[pl.ds(xrow0(t), _R)], st_sems[sx]
                )

                nslot = (u + 2) % _NX

                @pl.when(t >= 2)
                def _():
                    wait(st_sems, nslot, xb[nslot])

                @pl.when(t + 2 < n_steps)
                def _():
                    issue_x(t + 2, nslot)

        for t in (n_steps - 2, n_steps - 1):
            sx = t % _NX
            wait(st_sems, sx, xb[sx])

    out = sc_add(x2, pe)
    return out.reshape(batch, seq_len, d_model)


# speedup vs baseline: 1.1230x; 1.1230x over previous
import functools
import jax
import jax.numpy as jnp
from jax import lax
from jax.experimental import pallas as pl
from jax.experimental.pallas import tpu as pltpu
from jax.experimental.pallas import tpu_sc as plsc

_R = 16
_NX = 4
_NP = 2


def kernel(x, pe):
    batch, seq_len, d_model = x.shape
    n_rows = batch * seq_len
    x2 = x.reshape(n_rows, d_model)
    nw = 32
    seq_per_w = seq_len // nw
    n_chunks = seq_per_w // _R
    n_steps = n_chunks * batch

    mesh = plsc.VectorSubcoreMesh(
        core_axis_name="c", subcore_axis_name="s", num_cores=2, num_subcores=16
    )

    @functools.partial(
        pl.kernel,
        mesh=mesh,
        out_type=jax.ShapeDtypeStruct((n_rows, d_model), jnp.float32),
        scratch_types=[
            [pltpu.VMEM((_R, d_model), jnp.float32) for _ in range(_NX)],
            [pltpu.VMEM((_R, d_model), jnp.float32) for _ in range(_NP)],
            [pltpu.SemaphoreType.DMA for _ in range(_NX)],
            [pltpu.SemaphoreType.DMA for _ in range(_NP)],
            [pltpu.SemaphoreType.DMA for _ in range(_NX)],
        ],
    )
    def sc_add(x_hbm, pe_hbm, out_hbm, xb, peb, x_sems, pe_sems, st_sems):
        wid = lax.axis_index("s") * 2 + lax.axis_index("c")
        seq0 = wid * seq_per_w

        def xrow0(t):
            return (t % batch) * seq_len + seq0 + (t // batch) * _R

        def issue_x(t, sx):
            pltpu.async_copy(x_hbm.at[pl.ds(xrow0(t), _R)], xb[sx], x_sems[sx])

        def issue_pe(c, sp):
            pltpu.async_copy(
                pe_hbm.at[pl.ds(seq0 + c * _R, _R)], peb[sp], pe_sems[sp]
            )

        def wait(sems, slot, dst, src=None):
            pltpu.make_async_copy(
                x_hbm.at[pl.ds(0, _R)] if src is None else src, dst, sems[slot]
            ).wait()

        def compute(sx, sp):
            @pl.loop(0, _R)
            def row(r):
                @pl.loop(0, d_model // 16, unroll=8)
                def col(g):
                    sl = pl.ds(g * 16, 16)
                    plsc.addupdate(xb[sx].at[r, sl], peb[sp][r, sl])

        issue_pe(0, 0)
        issue_x(0, 0)
        issue_x(1, 1)

        @pl.loop(0, n_steps // 8)
        def outer(i):
            for u in range(8):
                t = i * 8 + u
                sx = u % _NX
                sp = u // 4
                c = t // batch

                nslot = (u + 2) % _NX

                @pl.when(t >= 2)
                def _():
                    wait(st_sems, nslot, xb[nslot])

                @pl.when(t + 2 < n_steps)
                def _():
                    issue_x(t + 2, nslot)

                if u % 4 == 0:
                    wait(pe_sems, sp, peb[sp])

                    @pl.when(c + 1 < n_chunks)
                    def _():
                        issue_pe(c + 1, (sp + 1) % _NP)

                wait(x_sems, sx, xb[sx])
                compute(sx, sp)
                pltpu.async_copy(
                    xb[sx], out_hbm.at[pl.ds(xrow0(t), _R)], st_sems[sx]
                )

        for t in (n_steps - 2, n_steps - 1):
            sx = t % _NX
            wait(st_sems, sx, xb[sx])

    out = sc_add(x2, pe)
    return out.reshape(batch, seq_len, d_model)


# speedup vs baseline: 3.5975x; 3.2034x over previous
import jax
import jax.numpy as jnp
from jax.experimental import pallas as pl


_BLK_S = 2048


def _pe_add_kernel(x_ref, pe_ref, o_ref):
    o_ref[...] = x_ref[...] + pe_ref[...]


def kernel(x, pe):
    batch, seq_len, d_model = x.shape
    num_s = seq_len // _BLK_S
    return pl.pallas_call(
        _pe_add_kernel,
        grid=(num_s, batch),
        in_specs=[
            pl.BlockSpec((1, _BLK_S, d_model), lambda s, b: (b, s, 0)),
            pl.BlockSpec((_BLK_S, d_model), lambda s, b: (s, 0)),
        ],
        out_specs=pl.BlockSpec((1, _BLK_S, d_model), lambda s, b: (b, s, 0)),
        out_shape=jax.ShapeDtypeStruct(x.shape, x.dtype),
    )(x, pe)
